# baseline (speedup 1.0000x reference)
import functools

import jax
import jax.numpy as jnp
from jax.experimental import pallas as pl
from jax.experimental.pallas import tpu as pltpu


def _vq_kernel(ninv, z_ref, cb_ref, zq_ref, idx_ref, loss_ref, cn_ref):
    b = pl.program_id(0)
    t = pl.program_id(1)

    zb = z_ref[0]
    cb = cb_ref[...]

    @pl.when(jnp.logical_and(b == 0, t == 0))
    def _precompute():
        cn_ref[...] = 0.5 * jnp.sum(cb * cb, axis=1, keepdims=True)

    half_cnorm = cn_ref[...]
    half_znorm = 0.5 * jnp.sum(zb * zb, axis=0, keepdims=True)
    mm = jax.lax.dot(cb, zb, preferred_element_type=jnp.float32)
    scores = (half_znorm + half_cnorm) - mm

    n = scores.shape[0]
    minval = jnp.min(scores, axis=0)
    row_iota = jax.lax.broadcasted_iota(jnp.int32, scores.shape, 0)
    widx = jnp.min(jnp.where(scores == minval[None, :], row_iota, n),
                   axis=0).astype(jnp.int32)
    idx_ref[0, 0, :] = widx

    onehot = (row_iota == widx[None, :]).astype(jnp.float32)
    zqb = jax.lax.dot_general(
        cb, onehot, (((0,), (0,)), ((), ())),
        preferred_element_type=jnp.float32)
    zq_ref[0] = zqb

    d = zqb - zb
    part = jnp.sum(d * d, keepdims=True).reshape(1, 1) * ninv

    @pl.when(jnp.logical_and(b == 0, t == 0))
    def _init():
        loss_ref[...] = jnp.zeros_like(part)

    loss_ref[...] += part


def kernel(z, codebook):
    B, C, T = z.shape
    N, _ = codebook.shape
    TT = 512
    grid = (B, T // TT)

    zq, idx3, loss = pl.pallas_call(
        functools.partial(_vq_kernel, 1.0 / float(z.size)),
        grid=grid,
        in_specs=[
            pl.BlockSpec((1, C, TT), lambda b, t: (b, 0, t)),
            pl.BlockSpec((N, C), lambda b, t: (0, 0)),
        ],
        out_specs=[
            pl.BlockSpec((1, C, TT), lambda b, t: (b, 0, t)),
            pl.BlockSpec((1, 1, TT), lambda b, t: (b, 0, t)),
            pl.BlockSpec((1, 1), lambda b, t: (0, 0)),
        ],
        out_shape=[
            jax.ShapeDtypeStruct((B, C, T), jnp.float32),
            jax.ShapeDtypeStruct((B, 1, T), jnp.int32),
            jax.ShapeDtypeStruct((1, 1), jnp.float32),
        ],
        scratch_shapes=[pltpu.VMEM((N, 1), jnp.float32)],
    )(z, codebook)

    return zq, idx3.reshape(B, T), loss[0, 0]


# speedup vs baseline: 1.1141x; 1.1141x over previous
import functools

import jax
import jax.numpy as jnp
from jax.experimental import pallas as pl
from jax.experimental.pallas import tpu as pltpu


def _vq_kernel(ninv, z_ref, cb_ref, zq_ref, idx_ref, loss_ref):
    b = pl.program_id(0)
    t = pl.program_id(1)

    zb = z_ref[0]
    cb = cb_ref[...]

    half_cnorm = 0.5 * jnp.sum(cb * cb, axis=1, keepdims=True)
    half_znorm = 0.5 * jnp.sum(zb * zb, axis=0, keepdims=True)
    mm = jax.lax.dot(cb, zb, preferred_element_type=jnp.float32)
    scores = (half_znorm + half_cnorm) - mm

    n = scores.shape[0]
    minval = jnp.min(scores, axis=0)
    row_iota = jax.lax.broadcasted_iota(jnp.int32, scores.shape, 0)
    widx = jnp.min(jnp.where(scores == minval[None, :], row_iota, n),
                   axis=0).astype(jnp.int32)
    idx_ref[0, 0, :] = widx

    onehot = (row_iota == widx[None, :]).astype(jnp.float32)
    zqb = jax.lax.dot_general(
        cb, onehot, (((0,), (0,)), ((), ())),
        preferred_element_type=jnp.float32)
    zq_ref[0] = zqb

    d = zqb - zb
    part = jnp.sum(d * d, keepdims=True).reshape(1, 1) * ninv

    @pl.when(jnp.logical_and(b == 0, t == 0))
    def _init():
        loss_ref[...] = jnp.zeros_like(part)

    loss_ref[...] += part


def kernel(z, codebook):
    B, C, T = z.shape
    N, _ = codebook.shape
    TT = 512
    grid = (B, T // TT)

    zq, idx3, loss = pl.pallas_call(
        functools.partial(_vq_kernel, 1.0 / float(z.size)),
        grid=grid,
        in_specs=[
            pl.BlockSpec((1, C, TT), lambda b, t: (b, 0, t)),
            pl.BlockSpec((N, C), lambda b, t: (0, 0)),
        ],
        out_specs=[
            pl.BlockSpec((1, C, TT), lambda b, t: (b, 0, t)),
            pl.BlockSpec((1, 1, TT), lambda b, t: (b, 0, t)),
            pl.BlockSpec((1, 1), lambda b, t: (0, 0)),
        ],
        out_shape=[
            jax.ShapeDtypeStruct((B, C, T), jnp.float32),
            jax.ShapeDtypeStruct((B, 1, T), jnp.int32),
            jax.ShapeDtypeStruct((1, 1), jnp.float32),
        ],
    )(z, codebook)

    return zq, idx3.reshape(B, T), loss[0, 0]


# speedup vs baseline: 1.3122x; 1.1779x over previous
import functools

import jax
import jax.numpy as jnp
from jax.experimental import pallas as pl
from jax.experimental.pallas import tpu as pltpu


def _vq_kernel(ninv, z_ref, cb_ref, zq_ref, idx_ref, loss_ref):
    b = pl.program_id(0)
    t = pl.program_id(1)

    zb = z_ref[0]
    cb = cb_ref[...]

    half_cnorm = 0.5 * jnp.sum(cb * cb, axis=1, keepdims=True)
    half_znorm = 0.5 * jnp.sum(zb * zb, axis=0, keepdims=True)
    mm = jax.lax.dot(cb, zb, preferred_element_type=jnp.float32)
    scores = (half_znorm + half_cnorm) - mm

    n = scores.shape[0]
    minval = jnp.min(scores, axis=0)
    row_iota = jax.lax.broadcasted_iota(jnp.int32, scores.shape, 0)
    widx = jnp.min(jnp.where(scores == minval[None, :], row_iota, n),
                   axis=0).astype(jnp.int32)
    idx_ref[0, 0, :] = widx

    onehot = (row_iota == widx[None, :]).astype(jnp.float32)
    zqb = jax.lax.dot_general(
        cb, onehot, (((0,), (0,)), ((), ())),
        preferred_element_type=jnp.float32)
    zq_ref[0] = zqb

    d = zqb - zb
    part = jnp.sum(d * d, keepdims=True).reshape(1, 1) * ninv

    @pl.when(jnp.logical_and(b == 0, t == 0))
    def _init():
        loss_ref[...] = jnp.zeros_like(part)

    loss_ref[...] += part


def kernel(z, codebook):
    B, C, T = z.shape
    N, _ = codebook.shape
    TT = 1024
    grid = (B, T // TT)

    zq, idx3, loss = pl.pallas_call(
        functools.partial(_vq_kernel, 1.0 / float(z.size)),
        grid=grid,
        in_specs=[
            pl.BlockSpec((1, C, TT), lambda b, t: (b, 0, t)),
            pl.BlockSpec((N, C), lambda b, t: (0, 0)),
        ],
        out_specs=[
            pl.BlockSpec((1, C, TT), lambda b, t: (b, 0, t)),
            pl.BlockSpec((1, 1, TT), lambda b, t: (b, 0, t)),
            pl.BlockSpec((1, 1), lambda b, t: (0, 0)),
        ],
        out_shape=[
            jax.ShapeDtypeStruct((B, C, T), jnp.float32),
            jax.ShapeDtypeStruct((B, 1, T), jnp.int32),
            jax.ShapeDtypeStruct((1, 1), jnp.float32),
        ],
    )(z, codebook)

    return zq, idx3.reshape(B, T), loss[0, 0]


# speedup vs baseline: 1.3872x; 1.0571x over previous
import functools

import jax
import jax.numpy as jnp
from jax.experimental import pallas as pl
from jax.experimental.pallas import tpu as pltpu


def _vq_kernel(ninv, nb, z_ref, cb_ref, zq_ref, idx_ref, loss_ref):
    b = pl.program_id(0)

    cb = cb_ref[...]

    half_cnorm = 0.5 * jnp.sum(cb * cb, axis=1, keepdims=True)

    part = jnp.zeros((1, 1), jnp.float32)
    for j in range(nb):
        zb = z_ref[j]
        half_znorm = 0.5 * jnp.sum(zb * zb, axis=0, keepdims=True)
        mm = jax.lax.dot(cb, zb, preferred_element_type=jnp.float32)
        scores = (half_znorm + half_cnorm) - mm

        n = scores.shape[0]
        minval = jnp.min(scores, axis=0)
        row_iota = jax.lax.broadcasted_iota(jnp.int32, scores.shape, 0)
        widx = jnp.min(jnp.where(scores == minval[None, :], row_iota, n),
                       axis=0).astype(jnp.int32)
        idx_ref[j, 0, :] = widx

        onehot = (row_iota == widx[None, :]).astype(jnp.float32)
        zqb = jax.lax.dot_general(
            cb, onehot, (((0,), (0,)), ((), ())),
            preferred_element_type=jnp.float32)
        zq_ref[j] = zqb

        d = zqb - zb
        part += jnp.sum(d * d, keepdims=True).reshape(1, 1) * ninv

    @pl.when(b == 0)
    def _init():
        loss_ref[...] = jnp.zeros_like(part)

    loss_ref[...] += part


def kernel(z, codebook):
    B, C, T = z.shape
    N, _ = codebook.shape
    NB = 2
    grid = (B // NB,)

    zq, idx3, loss = pl.pallas_call(
        functools.partial(_vq_kernel, 1.0 / float(z.size), NB),
        grid=grid,
        in_specs=[
            pl.BlockSpec((NB, C, T), lambda b: (b, 0, 0)),
            pl.BlockSpec((N, C), lambda b: (0, 0)),
        ],
        out_specs=[
            pl.BlockSpec((NB, C, T), lambda b: (b, 0, 0)),
            pl.BlockSpec((NB, 1, T), lambda b: (b, 0, 0)),
            pl.BlockSpec((1, 1), lambda b: (0, 0)),
        ],
        out_shape=[
            jax.ShapeDtypeStruct((B, C, T), jnp.float32),
            jax.ShapeDtypeStruct((B, 1, T), jnp.int32),
            jax.ShapeDtypeStruct((1, 1), jnp.float32),
        ],
    )(z, codebook)

    return zq, idx3.reshape(B, T), loss[0, 0]


# speedup vs baseline: 1.4258x; 1.0278x over previous
import functools

import jax
import jax.numpy as jnp
from jax.experimental import pallas as pl
from jax.experimental.pallas import tpu as pltpu


def _vq_kernel(ninv, nb, z_ref, cb_ref, zq_ref, idx_ref, loss_ref):
    b = pl.program_id(0)

    cb = cb_ref[...]

    half_cnorm = 0.5 * jnp.sum(cb * cb, axis=1, keepdims=True)

    part = jnp.zeros((1, 1), jnp.float32)
    for j in range(nb):
        zb = z_ref[j]
        half_znorm = 0.5 * jnp.sum(zb * zb, axis=0, keepdims=True)
        mm = jax.lax.dot(cb, zb, preferred_element_type=jnp.float32)
        scores = (half_znorm + half_cnorm) - mm

        n = scores.shape[0]
        minval = jnp.min(scores, axis=0)
        row_iota = jax.lax.broadcasted_iota(jnp.int32, scores.shape, 0)
        widx = jnp.min(jnp.where(scores == minval[None, :], row_iota, n),
                       axis=0).astype(jnp.int32)
        idx_ref[j, 0, :] = widx

        onehot = (row_iota == widx[None, :]).astype(jnp.float32)
        zqb = jax.lax.dot_general(
            cb, onehot, (((0,), (0,)), ((), ())),
            preferred_element_type=jnp.float32)
        zq_ref[j] = zqb

        d = zqb - zb
        part += jnp.sum(d * d, keepdims=True).reshape(1, 1) * ninv

    @pl.when(b == 0)
    def _init():
        loss_ref[...] = jnp.zeros_like(part)

    loss_ref[...] += part


def kernel(z, codebook):
    B, C, T = z.shape
    N, _ = codebook.shape
    NB = 4
    grid = (B // NB,)

    zq, idx3, loss = pl.pallas_call(
        functools.partial(_vq_kernel, 1.0 / float(z.size), NB),
        grid=grid,
        in_specs=[
            pl.BlockSpec((NB, C, T), lambda b: (b, 0, 0)),
            pl.BlockSpec((N, C), lambda b: (0, 0)),
        ],
        out_specs=[
            pl.BlockSpec((NB, C, T), lambda b: (b, 0, 0)),
            pl.BlockSpec((NB, 1, T), lambda b: (b, 0, 0)),
            pl.BlockSpec((1, 1), lambda b: (0, 0)),
        ],
        out_shape=[
            jax.ShapeDtypeStruct((B, C, T), jnp.float32),
            jax.ShapeDtypeStruct((B, 1, T), jnp.int32),
            jax.ShapeDtypeStruct((1, 1), jnp.float32),
        ],
    )(z, codebook)

    return zq, idx3.reshape(B, T), loss[0, 0]


# speedup vs baseline: 1.5053x; 1.0558x over previous
import functools

import jax
import jax.numpy as jnp
from jax.experimental import pallas as pl
from jax.experimental.pallas import tpu as pltpu


def _vq_kernel(ninv, nb, z_ref, cb_ref, zq_ref, idx_ref, loss_ref):
    b = pl.program_id(0)

    cb = cb_ref[...]

    half_cnorm = 0.5 * jnp.sum(cb * cb, axis=1, keepdims=True)

    part = jnp.zeros((1, 1), jnp.float32)
    for j in range(nb):
        zb = z_ref[j]
        half_znorm = 0.5 * jnp.sum(zb * zb, axis=0, keepdims=True)
        mm = jax.lax.dot(cb, zb, preferred_element_type=jnp.float32)
        scores = (half_znorm + half_cnorm) - mm

        n = scores.shape[0]
        minval = jnp.min(scores, axis=0)
        row_iota = jax.lax.broadcasted_iota(jnp.int32, scores.shape, 0)
        widx = jnp.min(jnp.where(scores == minval[None, :], row_iota, n),
                       axis=0).astype(jnp.int32)
        idx_ref[j, 0, :] = widx

        onehot = (row_iota == widx[None, :]).astype(jnp.float32)
        zqb = jax.lax.dot_general(
            cb, onehot, (((0,), (0,)), ((), ())),
            preferred_element_type=jnp.float32)
        zq_ref[j] = zqb

        part += jnp.sum(2.0 * minval, keepdims=True).reshape(1, 1) * ninv

    @pl.when(b == 0)
    def _init():
        loss_ref[...] = jnp.zeros_like(part)

    loss_ref[...] += part


def kernel(z, codebook):
    B, C, T = z.shape
    N, _ = codebook.shape
    NB = 4
    grid = (B // NB,)

    zq, idx3, loss = pl.pallas_call(
        functools.partial(_vq_kernel, 1.0 / float(z.size), NB),
        grid=grid,
        in_specs=[
            pl.BlockSpec((NB, C, T), lambda b: (b, 0, 0)),
            pl.BlockSpec((N, C), lambda b: (0, 0)),
        ],
        out_specs=[
            pl.BlockSpec((NB, C, T), lambda b: (b, 0, 0)),
            pl.BlockSpec((NB, 1, T), lambda b: (b, 0, 0)),
            pl.BlockSpec((1, 1), lambda b: (0, 0)),
        ],
        out_shape=[
            jax.ShapeDtypeStruct((B, C, T), jnp.float32),
            jax.ShapeDtypeStruct((B, 1, T), jnp.int32),
            jax.ShapeDtypeStruct((1, 1), jnp.float32),
        ],
    )(z, codebook)

    return zq, idx3.reshape(B, T), loss[0, 0]
